# baseline (speedup 1.0000x reference)
import jax
import jax.numpy as jnp
from jax.experimental import pallas as pl
from jax.experimental.pallas import tpu as pltpu

_B, _H, _W = 8, 512, 512
_N = _H * _W
_K = 4
_R = _H // _K


def _bce_kernel(p_ref, t_ref, out_ref, acc_ref):
    b = pl.program_id(0)
    k = pl.program_id(1)
    p = p_ref[0]
    t = t_ref[0]
    sel = jnp.where(t == 1.0, p, 1.0 - p)
    v = jnp.maximum(jnp.log(sel), -100.0)
    total = jnp.sum(v)
    s1 = jnp.sum(t * v)
    c1 = jnp.sum(t)

    @pl.when(k == 0)
    def _reset():
        acc_ref[0] = c1
        acc_ref[1] = s1
        acc_ref[2] = total

    @pl.when(k != 0)
    def _accum():
        acc_ref[0] += c1
        acc_ref[1] += s1
        acc_ref[2] += total

    @pl.when(jnp.logical_and(b == 0, k == 0))
    def _init_out():
        out_ref[:, :] = jnp.zeros((1, 1), jnp.float32)

    @pl.when(k == _K - 1)
    def _finalize():
        c1b = acc_ref[0]
        s1b = acc_ref[1]
        s0b = acc_ref[2] - s1b
        contrib = (c1b * s1b + (_N - c1b) * s0b) * (-1.0 / (_N * float(_N) * _B))
        out_ref[:, :] += jnp.full((1, 1), contrib)


def kernel(predict, target):
    out = pl.pallas_call(
        _bce_kernel,
        grid=(_B, _K),
        in_specs=[
            pl.BlockSpec((1, _R, _W), lambda b, k: (b, k, 0)),
            pl.BlockSpec((1, _R, _W), lambda b, k: (b, k, 0)),
        ],
        out_specs=pl.BlockSpec((1, 1), lambda b, k: (0, 0)),
        out_shape=jax.ShapeDtypeStruct((1, 1), jnp.float32),
        scratch_shapes=[pltpu.SMEM((4,), jnp.float32)],
    )(predict, target)
    return out[0, 0]


# speedup vs baseline: 2.1653x; 2.1653x over previous
import jax
import jax.numpy as jnp
from jax.experimental import pallas as pl

_B, _H, _W = 8, 512, 512
_N = _H * _W


def _bce_kernel(p_ref, t_ref, out_ref):
    b = pl.program_id(0)
    p = p_ref[0]
    t = t_ref[0]
    sel = jnp.where(t == 1.0, p, 1.0 - p)
    v = jnp.maximum(jnp.log(sel), -100.0)
    total = jnp.sum(v)
    s1 = jnp.sum(t * v)
    c1 = jnp.sum(t)
    s0 = total - s1
    contrib = (c1 * s1 + (_N - c1) * s0) * (-1.0 / (_N * float(_N) * _B))

    @pl.when(b == 0)
    def _init():
        out_ref[:, :] = jnp.zeros((1, 1), jnp.float32)

    out_ref[:, :] += jnp.full((1, 1), contrib)


def kernel(predict, target):
    out = pl.pallas_call(
        _bce_kernel,
        grid=(_B,),
        in_specs=[
            pl.BlockSpec((1, _H, _W), lambda b: (b, 0, 0)),
            pl.BlockSpec((1, _H, _W), lambda b: (b, 0, 0)),
        ],
        out_specs=pl.BlockSpec((1, 1), lambda b: (0, 0)),
        out_shape=jax.ShapeDtypeStruct((1, 1), jnp.float32),
    )(predict, target)
    return out[0, 0]


# speedup vs baseline: 2.7428x; 1.2667x over previous
import jax
import jax.numpy as jnp
from jax.experimental import pallas as pl

_B, _H, _W = 8, 512, 512
_N = _H * _W
_BB = 2
_STEPS = _B // _BB


def _bce_kernel(p_ref, t_ref, out_ref):
    b = pl.program_id(0)
    p = p_ref[...]
    t = t_ref[...]
    sel = jnp.where(t == 1.0, p, 1.0 - p)
    v = jnp.maximum(jnp.log(sel), -100.0)
    totalv = jnp.sum(v, axis=(1, 2))
    s1v = jnp.sum(t * v, axis=(1, 2))
    c1v = jnp.sum(t, axis=(1, 2))
    s0v = totalv - s1v
    contrib = jnp.sum(c1v * s1v + (_N - c1v) * s0v) * (
        -1.0 / (_N * float(_N) * _B)
    )

    @pl.when(b == 0)
    def _init():
        out_ref[:, :] = jnp.zeros((1, 1), jnp.float32)

    out_ref[:, :] += jnp.full((1, 1), contrib)


def kernel(predict, target):
    out = pl.pallas_call(
        _bce_kernel,
        grid=(_STEPS,),
        in_specs=[
            pl.BlockSpec((_BB, _H, _W), lambda b: (b, 0, 0)),
            pl.BlockSpec((_BB, _H, _W), lambda b: (b, 0, 0)),
        ],
        out_specs=pl.BlockSpec((1, 1), lambda b: (0, 0)),
        out_shape=jax.ShapeDtypeStruct((1, 1), jnp.float32),
    )(predict, target)
    return out[0, 0]


# speedup vs baseline: 2.7760x; 1.0121x over previous
import jax
import jax.numpy as jnp
from jax.experimental import pallas as pl

_B, _H, _W = 8, 512, 512
_N = _H * _W
_BB = 4
_STEPS = _B // _BB


def _bce_kernel(p_ref, t_ref, out_ref):
    b = pl.program_id(0)
    p = p_ref[...]
    t = t_ref[...]
    sel = jnp.where(t == 1.0, p, 1.0 - p)
    v = jnp.maximum(jnp.log(sel), -100.0)
    totalv = jnp.sum(v, axis=(1, 2))
    s1v = jnp.sum(t * v, axis=(1, 2))
    c1v = jnp.sum(t, axis=(1, 2))
    s0v = totalv - s1v
    contrib = jnp.sum(c1v * s1v + (_N - c1v) * s0v) * (
        -1.0 / (_N * float(_N) * _B)
    )

    @pl.when(b == 0)
    def _init():
        out_ref[:, :] = jnp.zeros((1, 1), jnp.float32)

    out_ref[:, :] += jnp.full((1, 1), contrib)


def kernel(predict, target):
    out = pl.pallas_call(
        _bce_kernel,
        grid=(_STEPS,),
        in_specs=[
            pl.BlockSpec((_BB, _H, _W), lambda b: (b, 0, 0)),
            pl.BlockSpec((_BB, _H, _W), lambda b: (b, 0, 0)),
        ],
        out_specs=pl.BlockSpec((1, 1), lambda b: (0, 0)),
        out_shape=jax.ShapeDtypeStruct((1, 1), jnp.float32),
    )(predict, target)
    return out[0, 0]
